# baseline (speedup 1.0000x reference)
import functools

import jax
import jax.numpy as jnp
from jax import lax
from jax.experimental import pallas as pl
from jax.experimental.pallas import tpu as pltpu, tpu_sc as plsc

S, C, L, T, B = 10, 100, 4, 6, 16384
R = S * C
LT = L * T
TABLE_N = R * LT

_info = plsc.get_sparse_core_info()
_NC, _NS, _LANES = _info.num_cores, _info.num_subcores, _info.num_lanes
_NW = _NC * _NS
_BPW = B // _NW
_STEPS = _BPW // _LANES

_RPT = 64
_FPT = _RPT * LT
_FSTEPS = _FPT // _LANES
_LAST_ROW0 = R - _RPT

_mesh = plsc.VectorSubcoreMesh(core_axis_name="c", subcore_axis_name="s")


@functools.partial(
    pl.kernel,
    mesh=_mesh,
    out_type=(
        jax.ShapeDtypeStruct((B,), jnp.float32),
        jax.ShapeDtypeStruct((_NC * TABLE_N,), jnp.float32),
    ),
    compiler_params=pltpu.CompilerParams(needs_layout_passes=False),
    scratch_types=[
        pltpu.VMEM((_RPT, L, T), jnp.float32),
        pltpu.VMEM((_FPT,), jnp.float32),
        pltpu.VMEM((_BPW,), jnp.int32),
        pltpu.VMEM((_BPW,), jnp.int32),
        pltpu.VMEM((_BPW,), jnp.int32),
        pltpu.VMEM((_BPW,), jnp.int32),
        pltpu.VMEM((_BPW,), jnp.int32),
        pltpu.VMEM((_BPW,), jnp.float32),
        pltpu.SemaphoreType.DMA,
    ],
)
def _balancer_gather(table_hbm, src_hbm, cnt_hbm, lab_hbm, vt_hbm,
                     out_hbm, flat_hbm,
                     chunk_v, flat_v, src_v, cnt_v, lab_v, vt_v, lin_v, out_v,
                     sem):
    cid = lax.axis_index("c")
    sid = lax.axis_index("s")
    wid = sid * _NC + cid
    base = wid * _BPW

    row0 = jnp.minimum(sid * _RPT, _LAST_ROW0)

    sl_in = pl.ds(base, _BPW)
    copies = [
        pltpu.async_copy(table_hbm.at[pl.ds(row0, _RPT)], chunk_v, sem),
        pltpu.async_copy(src_hbm.at[sl_in], src_v, sem),
        pltpu.async_copy(cnt_hbm.at[sl_in], cnt_v, sem),
        pltpu.async_copy(lab_hbm.at[sl_in], lab_v, sem),
        pltpu.async_copy(vt_hbm.at[sl_in], vt_v, sem),
    ]
    copies[0].wait()

    for j in range(_FSTEPS):
        f = lax.iota(jnp.int32, _LANES) + j * _LANES
        rr = f // LT
        l = (f % LT) // T
        t = f % T
        sl = pl.ds(j * _LANES, _LANES)
        flat_v[sl] = plsc.load_gather(chunk_v, [rr, l, t])

    flat_off = cid * TABLE_N + row0 * LT
    wr = pltpu.async_copy(flat_v, flat_hbm.at[pl.ds(flat_off, _FPT)], sem)

    for cp in copies[1:]:
        cp.wait()
    core_base = cid * TABLE_N
    for i in range(_STEPS):
        sl = pl.ds(i * _LANES, _LANES)
        lin_v[sl] = (core_base + src_v[sl] * (C * LT) + cnt_v[sl] * LT
                     + lab_v[sl] * T + vt_v[sl])

    wr.wait()
    plsc.subcore_barrier()

    pltpu.async_copy(flat_hbm.at[lin_v], out_v, sem).wait()
    pltpu.sync_copy(out_v, out_hbm.at[pl.ds(base, _BPW)])


def kernel(label_balancing_weights_sclt, sources, alt_counts, labels, variant_types):
    table = label_balancing_weights_sclt.reshape(R, L, T)
    out, _ = _balancer_gather(table, sources, alt_counts, labels, variant_types)
    return out


# speedup vs baseline: 1.1327x; 1.1327x over previous
import functools

import jax
import jax.numpy as jnp
from jax import lax
from jax.experimental import pallas as pl
from jax.experimental.pallas import tpu as pltpu, tpu_sc as plsc

S, C, L, T, B = 10, 100, 4, 6, 16384
R = S * C
TABLE_N = R * L * T

_info = plsc.get_sparse_core_info()
_NC, _NS, _LANES = _info.num_cores, _info.num_subcores, _info.num_lanes
_NW = _NC * _NS
_BPW = B // _NW
_STEPS = _BPW // _LANES

_RPT = 64
_LAST_ROW0 = R - _RPT

_mesh = plsc.VectorSubcoreMesh(core_axis_name="c", subcore_axis_name="s")


@functools.partial(
    pl.kernel,
    mesh=_mesh,
    out_type=jax.ShapeDtypeStruct((B,), jnp.float32),
    compiler_params=pltpu.CompilerParams(needs_layout_passes=False),
    scratch_types=[
        pltpu.VMEM_SHARED((TABLE_N,), jnp.float32),
        pltpu.VMEM((_RPT, L, T), jnp.float32),
        pltpu.VMEM((L * T * _RPT,), jnp.float32),
        pltpu.VMEM((_BPW,), jnp.int32),
        pltpu.VMEM((_BPW,), jnp.int32),
        pltpu.VMEM((_BPW,), jnp.int32),
        pltpu.VMEM((_BPW,), jnp.int32),
        pltpu.VMEM((_BPW,), jnp.int32),
        pltpu.VMEM((_BPW,), jnp.float32),
        pltpu.SemaphoreType.DMA,
    ],
)
def _balancer_gather(table_hbm, src_hbm, cnt_hbm, lab_hbm, vt_hbm, out_hbm,
                     shared_v, chunk_v, flat_v, src_v, cnt_v, lab_v, vt_v,
                     lin_v, out_v, sem):
    cid = lax.axis_index("c")
    sid = lax.axis_index("s")
    wid = sid * _NC + cid
    base = wid * _BPW

    row0 = jnp.minimum(sid * _RPT, _LAST_ROW0)

    sl_in = pl.ds(base, _BPW)
    copies = [
        pltpu.async_copy(table_hbm.at[pl.ds(row0, _RPT)], chunk_v, sem),
        pltpu.async_copy(src_hbm.at[sl_in], src_v, sem),
        pltpu.async_copy(cnt_hbm.at[sl_in], cnt_v, sem),
        pltpu.async_copy(lab_hbm.at[sl_in], lab_v, sem),
        pltpu.async_copy(vt_hbm.at[sl_in], vt_v, sem),
    ]
    copies[0].wait()

    lane_iota = lax.iota(jnp.int32, _LANES)
    for lt in range(L * T):
        l, t = lt // T, lt % T
        for k in range(_RPT // _LANES):
            rr = lane_iota + k * _LANES
            flat_v[pl.ds(lt * _RPT + k * _LANES, _LANES)] = plsc.load_gather(
                chunk_v, [rr, jnp.full_like(rr, l), jnp.full_like(rr, t)])

    fan = [
        pltpu.async_copy(
            flat_v.at[pl.ds(lt * _RPT, _RPT)],
            shared_v.at[pl.ds(lt * R + row0, _RPT)],
            sem,
        )
        for lt in range(L * T)
    ]

    for cp in copies[1:]:
        cp.wait()
    for i in range(_STEPS):
        sl = pl.ds(i * _LANES, _LANES)
        lin_v[sl] = ((lab_v[sl] * T + vt_v[sl]) * R
                     + src_v[sl] * C + cnt_v[sl])

    for cp in fan:
        cp.wait()
    plsc.subcore_barrier()

    pltpu.async_copy(shared_v.at[lin_v], out_v, sem).wait()
    pltpu.sync_copy(out_v, out_hbm.at[pl.ds(base, _BPW)])


def kernel(label_balancing_weights_sclt, sources, alt_counts, labels, variant_types):
    table = label_balancing_weights_sclt.reshape(R, L, T)
    return _balancer_gather(table, sources, alt_counts, labels, variant_types)
